# baseline (speedup 1.0000x reference)
import functools

import jax
import jax.numpy as jnp
from jax import lax
from jax.experimental import pallas as pl
from jax.experimental.pallas import tpu as pltpu
from jax.experimental.pallas import tpu_sc as plsc


def _make_sc_embed(batch: int, seq: int, embed: int):
    info = plsc.get_sparse_core_info()
    nc, ns = info.num_cores, info.num_subcores
    nw = nc * ns
    num_rows = batch * seq
    assert num_rows % nw == 0
    rows_per_w = num_rows // nw
    assert seq % rows_per_w == 0
    chunks_per_batch = seq // rows_per_w
    nchunk = 1
    assert rows_per_w % nchunk == 0
    crows = rows_per_w // nchunk
    mesh = plsc.VectorSubcoreMesh(core_axis_name="c", subcore_axis_name="s")

    @functools.partial(
        pl.kernel,
        mesh=mesh,
        out_type=jax.ShapeDtypeStruct((batch, seq, embed), jnp.float32),
        scratch_types=[
            pltpu.VMEM((rows_per_w,), jnp.int32),
            pltpu.VMEM((rows_per_w, embed), jnp.float32),
            pltpu.SemaphoreType.DMA,
        ]
        + [pltpu.SemaphoreType.DMA] * (2 * nchunk),
    )
    def sc_embed(x_hbm, tok_hbm, pos_hbm, out_hbm, idx_v, rows_v, psem, *sems):
        gsem = sems[0:nchunk]
        wsem = sems[nchunk : 2 * nchunk]
        wid = lax.axis_index("s") * nc + lax.axis_index("c")
        b = wid // chunks_per_batch
        s0 = (wid % chunks_per_batch) * rows_per_w

        pos_cp = pltpu.async_copy(pos_hbm.at[pl.ds(s0, rows_per_w)], rows_v, psem)
        pltpu.sync_copy(x_hbm.at[b, pl.ds(s0, rows_per_w)], idx_v)
        pos_cp.wait()

        g_cp = [
            pltpu.async_copy(
                tok_hbm.at[idx_v.at[pl.ds(c * crows, crows)]],
                rows_v.at[pl.ds(c * crows, crows)],
                gsem[c],
                add=True,
            )
            for c in range(nchunk)
        ]
        w_cp = []
        for c in range(nchunk):
            g_cp[c].wait()
            w_cp.append(
                pltpu.async_copy(
                    rows_v.at[pl.ds(c * crows, crows)],
                    out_hbm.at[b, pl.ds(s0 + c * crows, crows)],
                    wsem[c],
                )
            )
        for c in range(nchunk):
            w_cp[c].wait()

    return sc_embed


def kernel(x, token_table, pos_table):
    b, s = x.shape
    embed = token_table.shape[1]
    fn = _make_sc_embed(b, s, embed)
    return fn(x, token_table, pos_table)


# speedup vs baseline: 1.0077x; 1.0077x over previous
import functools

import jax
import jax.numpy as jnp
from jax import lax
from jax.experimental import pallas as pl
from jax.experimental.pallas import tpu as pltpu
from jax.experimental.pallas import tpu_sc as plsc


def _make_sc_embed(batch: int, seq: int, embed: int):
    info = plsc.get_sparse_core_info()
    nc, ns = info.num_cores, info.num_subcores
    nw = nc * ns
    num_rows = batch * seq
    assert num_rows % nw == 0
    rows_per_w = num_rows // nw
    assert seq % rows_per_w == 0
    chunks_per_batch = seq // rows_per_w
    nchunk = 2
    assert rows_per_w % nchunk == 0
    crows = rows_per_w // nchunk
    mesh = plsc.VectorSubcoreMesh(core_axis_name="c", subcore_axis_name="s")

    @functools.partial(
        pl.kernel,
        mesh=mesh,
        out_type=jax.ShapeDtypeStruct((batch, seq, embed), jnp.float32),
        scratch_types=[
            pltpu.VMEM((rows_per_w,), jnp.int32),
            pltpu.VMEM((rows_per_w, embed), jnp.float32),
            pltpu.SemaphoreType.DMA,
        ]
        + [pltpu.SemaphoreType.DMA] * (2 * nchunk),
    )
    def sc_embed(x_hbm, tok_hbm, pos_hbm, out_hbm, idx_v, rows_v, psem, *sems):
        gsem = sems[0:nchunk]
        wsem = sems[nchunk : 2 * nchunk]
        wid = lax.axis_index("s") * nc + lax.axis_index("c")
        b = wid // chunks_per_batch
        s0 = (wid % chunks_per_batch) * rows_per_w

        pos_cp = pltpu.async_copy(pos_hbm.at[pl.ds(s0, rows_per_w)], rows_v, psem)
        pltpu.sync_copy(x_hbm.at[b, pl.ds(s0, rows_per_w)], idx_v)
        pos_cp.wait()

        g_cp = [
            pltpu.async_copy(
                tok_hbm.at[idx_v.at[pl.ds(c * crows, crows)]],
                rows_v.at[pl.ds(c * crows, crows)],
                gsem[c],
                add=True,
            )
            for c in range(nchunk)
        ]
        w_cp = []
        for c in range(nchunk):
            g_cp[c].wait()
            w_cp.append(
                pltpu.async_copy(
                    rows_v.at[pl.ds(c * crows, crows)],
                    out_hbm.at[b, pl.ds(s0 + c * crows, crows)],
                    wsem[c],
                )
            )
        for c in range(nchunk):
            w_cp[c].wait()

    return sc_embed


def kernel(x, token_table, pos_table):
    b, s = x.shape
    embed = token_table.shape[1]
    fn = _make_sc_embed(b, s, embed)
    return fn(x, token_table, pos_table)
